# baseline (speedup 1.0000x reference)
import functools

import jax
import jax.numpy as jnp
from jax.experimental import pallas as pl
from jax.experimental.pallas import tpu as pltpu

TILE_N = 1024


def _moe_half(x, wg, bg, w1c, b1c, w2c, b2, gamma, beta, n_experts):
    logits = jnp.dot(x, wg, preferred_element_type=jnp.float32)
    logits = logits + bg
    iota_e = jax.lax.broadcasted_iota(jnp.int32, logits.shape, 1)
    m1 = jnp.max(logits, axis=1, keepdims=True)
    idx1 = jnp.min(jnp.where(logits == m1, iota_e, n_experts), axis=1,
                   keepdims=True)
    l2 = jnp.where(iota_e == idx1, -jnp.inf, logits)
    m2 = jnp.max(l2, axis=1, keepdims=True)
    idx2 = jnp.min(jnp.where(l2 == m2, iota_e, n_experts), axis=1,
                   keepdims=True)
    e2 = jnp.exp(m2 - m1)
    g1 = 1.0 / (1.0 + e2)
    g2 = e2 * g1
    gate_s = g1 * (iota_e == idx1) + g2 * (iota_e == idx2)
    sg_sum = jnp.sum(gate_s, axis=0, keepdims=True)

    xb = x.astype(jnp.bfloat16)
    h = jnp.dot(xb, w1c, preferred_element_type=jnp.float32)
    hb = h.astype(jnp.bfloat16)
    hb = jnp.maximum(hb + b1c, jnp.bfloat16(0.0))
    h_dim = hb.shape[1] // n_experts
    chunks = []
    for e in range(n_experts):
        ge = (g1 * (idx1 == e) + g2 * (idx2 == e)).astype(jnp.bfloat16)
        chunks.append(hb[:, e * h_dim:(e + 1) * h_dim] * ge)
    gh = jnp.concatenate(chunks, axis=1)
    moe = jnp.dot(gh, w2c, preferred_element_type=jnp.float32)
    moe = moe + jnp.dot(gate_s, b2, preferred_element_type=jnp.float32)

    y = moe + x
    mu = jnp.mean(y, axis=1, keepdims=True)
    yc = y - mu
    var = jnp.mean(yc * yc, axis=1, keepdims=True)
    out = yc * jax.lax.rsqrt(var + 1e-5) * gamma + beta
    return out, sg_sum


def _moe_body(x_ref, wg_ref, bg_ref, w1c_ref, b1c_ref, w2c_ref, b2_ref,
              gamma_ref, beta_ref, out_ref, lb_ref, gsum_ref,
              *, nt, n_experts, n_tokens, n_halves):
    t = pl.program_id(0)
    half = x_ref.shape[0] // n_halves
    args = (wg_ref[...], bg_ref[...], w1c_ref[...], b1c_ref[...],
            w2c_ref[...], b2_ref[...], gamma_ref[...], beta_ref[...])
    sg_total = None
    for p in range(n_halves):
        sl = slice(p * half, (p + 1) * half)
        out, sg = _moe_half(x_ref[sl, :], *args, n_experts)
        out_ref[sl, :] = out
        sg_total = sg if sg_total is None else sg_total + sg

    prev = jnp.where(t == 0, jnp.zeros_like(sg_total), gsum_ref[...])
    gsum_ref[...] = prev + sg_total

    @pl.when(t == nt - 1)
    def _():
        d_i = gsum_ref[...] / n_tokens
        lb_ref[...] = jnp.sum(d_i * jnp.log(d_i + 1e-8), keepdims=True
                              ).reshape(1, 1)


def kernel(x, W_gate, b_gate, W1, b1, W2, b2, gamma, beta):
    n, d = x.shape
    e_num = W_gate.shape[1]
    h_dim = W1.shape[2]
    eh = e_num * h_dim
    nt = n // TILE_N

    w1c = jnp.transpose(W1, (1, 0, 2)).reshape(d, eh).astype(jnp.bfloat16)
    b1c = b1.reshape(1, eh).astype(jnp.bfloat16)
    w2c = W2.reshape(eh, d).astype(jnp.bfloat16)

    body = functools.partial(_moe_body, nt=nt, n_experts=e_num, n_tokens=n, n_halves=4)
    const = lambda t: (0, 0)
    out, lb = pl.pallas_call(
        body,
        grid=(nt,),
        in_specs=[
            pl.BlockSpec((TILE_N, d), lambda t: (t, 0)),
            pl.BlockSpec((d, e_num), const),
            pl.BlockSpec((1, e_num), const),
            pl.BlockSpec((d, eh), const),
            pl.BlockSpec((1, eh), const),
            pl.BlockSpec((eh, d), const),
            pl.BlockSpec((e_num, d), const),
            pl.BlockSpec((1, d), const),
            pl.BlockSpec((1, d), const),
        ],
        out_specs=[
            pl.BlockSpec((TILE_N, d), lambda t: (t, 0)),
            pl.BlockSpec((1, 1), const),
        ],
        out_shape=[
            jax.ShapeDtypeStruct((n, d), jnp.float32),
            jax.ShapeDtypeStruct((1, 1), jnp.float32),
        ],
        scratch_shapes=[
            pltpu.VMEM((1, e_num), jnp.float32),
        ],
    )(x, W_gate, b_gate.reshape(1, e_num), w1c, b1c, w2c, b2,
      gamma.reshape(1, d), beta.reshape(1, d))
    return out, lb[0, 0]


# speedup vs baseline: 1.0036x; 1.0036x over previous
import functools

import jax
import jax.numpy as jnp
from jax.experimental import pallas as pl
from jax.experimental.pallas import tpu as pltpu

TILE_N = 1024


def _moe_half(x, wg, bg, w1c, b1c, w2c, b2, gamma, beta, n_experts):
    logits = jnp.dot(x, wg, preferred_element_type=jnp.float32)
    logits = logits + bg
    iota_e = jax.lax.broadcasted_iota(jnp.int32, logits.shape, 1)
    m1 = jnp.max(logits, axis=1, keepdims=True)
    idx1 = jnp.min(jnp.where(logits == m1, iota_e, n_experts), axis=1,
                   keepdims=True)
    l2 = jnp.where(iota_e == idx1, -jnp.inf, logits)
    m2 = jnp.max(l2, axis=1, keepdims=True)
    idx2 = jnp.min(jnp.where(l2 == m2, iota_e, n_experts), axis=1,
                   keepdims=True)
    e2 = jnp.exp(m2 - m1)
    g1 = 1.0 / (1.0 + e2)
    g2 = e2 * g1
    gate_s = g1 * (iota_e == idx1) + g2 * (iota_e == idx2)
    sg_sum = jnp.sum(gate_s, axis=0, keepdims=True)

    xb = x.astype(jnp.bfloat16)
    h = jnp.dot(xb, w1c, preferred_element_type=jnp.float32)
    hb = h.astype(jnp.bfloat16)
    hb = jnp.maximum(hb + b1c, jnp.bfloat16(0.0))
    h_dim = hb.shape[1] // n_experts
    chunks = []
    for e in range(n_experts):
        ge = (g1 * (idx1 == e) + g2 * (idx2 == e)).astype(jnp.bfloat16)
        chunks.append(hb[:, e * h_dim:(e + 1) * h_dim] * ge)
    gh = jnp.concatenate(chunks, axis=1)
    moe = jnp.dot(gh, w2c, preferred_element_type=jnp.float32)
    moe = moe + jnp.dot(gate_s, b2, preferred_element_type=jnp.float32)

    y = moe + x
    mu = jnp.mean(y, axis=1, keepdims=True)
    yc = y - mu
    var = jnp.mean(yc * yc, axis=1, keepdims=True)
    out = yc * jax.lax.rsqrt(var + 1e-5) * gamma + beta
    return out, sg_sum


def _moe_body(x_ref, wg_ref, bg_ref, w1c_ref, b1c_ref, w2c_ref, b2_ref,
              gamma_ref, beta_ref, out_ref, lb_ref, gsum_ref,
              *, nt, n_experts, n_tokens, n_halves):
    t = pl.program_id(0)
    half = x_ref.shape[0] // n_halves
    args = (wg_ref[...], bg_ref[...], w1c_ref[...], b1c_ref[...],
            w2c_ref[...], b2_ref[...], gamma_ref[...], beta_ref[...])
    sg_total = None
    for p in range(n_halves):
        sl = slice(p * half, (p + 1) * half)
        out, sg = _moe_half(x_ref[sl, :], *args, n_experts)
        out_ref[sl, :] = out
        sg_total = sg if sg_total is None else sg_total + sg

    prev = jnp.where(t == 0, jnp.zeros_like(sg_total), gsum_ref[...])
    gsum_ref[...] = prev + sg_total

    @pl.when(t == nt - 1)
    def _():
        d_i = gsum_ref[...] / n_tokens
        lb_ref[...] = jnp.sum(d_i * jnp.log(d_i + 1e-8), keepdims=True
                              ).reshape(1, 1)


def kernel(x, W_gate, b_gate, W1, b1, W2, b2, gamma, beta):
    n, d = x.shape
    e_num = W_gate.shape[1]
    h_dim = W1.shape[2]
    eh = e_num * h_dim
    nt = n // TILE_N

    w1c = jnp.transpose(W1, (1, 0, 2)).reshape(d, eh).astype(jnp.bfloat16)
    b1c = b1.reshape(1, eh).astype(jnp.bfloat16)
    w2c = W2.reshape(eh, d).astype(jnp.bfloat16)

    body = functools.partial(_moe_body, nt=nt, n_experts=e_num, n_tokens=n, n_halves=2)
    const = lambda t: (0, 0)
    out, lb = pl.pallas_call(
        body,
        grid=(nt,),
        in_specs=[
            pl.BlockSpec((TILE_N, d), lambda t: (t, 0)),
            pl.BlockSpec((d, e_num), const),
            pl.BlockSpec((1, e_num), const),
            pl.BlockSpec((d, eh), const),
            pl.BlockSpec((1, eh), const),
            pl.BlockSpec((eh, d), const),
            pl.BlockSpec((e_num, d), const),
            pl.BlockSpec((1, d), const),
            pl.BlockSpec((1, d), const),
        ],
        out_specs=[
            pl.BlockSpec((TILE_N, d), lambda t: (t, 0)),
            pl.BlockSpec((1, 1), const),
        ],
        out_shape=[
            jax.ShapeDtypeStruct((n, d), jnp.float32),
            jax.ShapeDtypeStruct((1, 1), jnp.float32),
        ],
        scratch_shapes=[
            pltpu.VMEM((1, e_num), jnp.float32),
        ],
    )(x, W_gate, b_gate.reshape(1, e_num), w1c, b1c, w2c, b2,
      gamma.reshape(1, d), beta.reshape(1, d))
    return out, lb[0, 0]
